# baseline (speedup 1.0000x reference)
import jax
import jax.numpy as jnp
from jax.experimental import pallas as pl
from jax.experimental.pallas import tpu as pltpu

DEPTH = 3
AF = 82
BF = 6
H = 300
BIN = 11
N = 100
M = 120
NB = 10
C = 5
NP = 104
RK = NB * NP
NNP = N * NP
K = 20
NEG = -3e38

_CHUNKS = [(x0, 8) for x0 in range(0, 96, 8)] + [(96, 4)]


def _dot(a, b):
    return jnp.dot(a, b, preferred_element_type=jnp.float32)


def _iota(shape, dim):
    return jax.lax.broadcasted_iota(jnp.int32, shape, dim)


def _egather(oh, tbl):
    bits = jax.lax.bitcast_convert_type(tbl, jnp.int32)
    acc = None
    for sh in (0, 8, 16, 24):
        byte = jnp.right_shift(bits, sh) & 0xFF
        plane = _dot(oh, byte.astype(jnp.float32)).astype(jnp.int32)
        term = jnp.left_shift(plane, sh)
        acc = term if acc is None else acc | term
    return jax.lax.bitcast_convert_type(acc, jnp.float32)


def _band_onehot(nbands):
    rows = nbands * NP
    r = _iota((rows, nbands), 0)
    c = _iota((rows, nbands), 1)
    return ((r >= c * NP) & (r < c * NP + NP)).astype(jnp.float32)


def _band_sum(tp, off):
    acc = tp[off:off + 8, :]
    for j in range(1, 13):
        acc = acc + tp[off + 8 * j:off + 8 * (j + 1), :]
    for st in (4, 2, 1):
        acc = acc[:st, :] + acc[st:2 * st, :]
    return acc


def _body(fat_r, fbd_r, anb_r, bnb_r, bf_r, bl_r, mn_r, ma_r,
          We_r, be_r, Wna_r, Wnb_r, Ws_r, Wu2_r, Wu1_r,
          Waa_r, Wab_r, Was_r, Wpl_r, Wpg_r, Wpb_r, bp_r, Wsc_r, bsc_r,
          out_r, tk_r):
    f32 = jnp.float32
    fat = fat_r[0]
    fbd = fbd_r[0]
    anb = anb_r[0]
    bnb = bnb_r[0]
    mnei = mn_r[0]
    mat = ma_r[0]

    oh_a = (anb == _iota((RK, N), 1)).astype(f32)
    oh_b = (bnb == _iota((RK, M), 1)).astype(f32)
    fb = _egather(oh_b, fbd)

    atom_feat = jnp.maximum(_dot(fat, We_r[...]) + be_r[...], 0.0)
    local = atom_feat
    for d in range(DEPTH):
        fnei = _egather(oh_a, atom_feat)
        hm = _dot(fnei, Wna_r[d]) * _dot(fb, Wnb_r[d]) * mnei
        f_nei = hm[0:N, :]
        for k in range(1, NB):
            f_nei = f_nei + hm[k * NP:k * NP + N, :]
        f_self = _dot(atom_feat, Ws_r[d])
        local = f_nei * f_self * mat
        npm = jnp.maximum(_dot(jnp.concatenate([fnei, fb], axis=1),
                               Wu2_r[d]), 0.0) * mnei
        nei_partial = npm[0:N, :]
        for k in range(1, NB):
            nei_partial = nei_partial + npm[k * NP:k * NP + N, :]
        atom_feat = jnp.maximum(
            _dot(jnp.concatenate([atom_feat, nei_partial], axis=1),
                 Wu1_r[d]), 0.0)

    a = _dot(local, Waa_r[...])
    zpad = jnp.zeros((NP - N, H), f32)
    a104 = jnp.concatenate([a, zpad], axis=0)
    l104 = jnp.concatenate([local, zpad], axis=0)
    boh = {nb: _band_onehot(nb) for nb in (8, 4)}
    ay = {nb: jnp.concatenate([a104] * nb, axis=0) for nb in (8, 4)}
    lt = {nb: jnp.concatenate([l104] * nb, axis=0) for nb in (8, 4)}

    gf_parts = []
    for x0, nb in _CHUNKS:
        rows = nb * NP
        ax = _egather(boh[nb], a[x0:x0 + nb, :])
        bfc = bf_r[0, x0 * NP:(x0 + nb) * NP, :]
        hid = jnp.maximum(_dot(bfc, Wab_r[...]) + (ax + ay[nb]), 0.0)
        logit = _dot(hid, Was_r[...])
        s = 1.0 / (1.0 + jnp.exp(-logit))
        tp = s * lt[nb]
        for b in range(nb):
            gf_parts.append(_band_sum(tp, b * NP))
    gf = jnp.concatenate(gf_parts, axis=0)
    g104 = jnp.concatenate([gf, zpad], axis=0)
    gy = {nb: jnp.concatenate([g104] * nb, axis=0) for nb in (8, 4)}

    for x0, nb in _CHUNKS:
        lp = _egather(boh[nb], local[x0:x0 + nb, :]) + lt[nb]
        gp = _egather(boh[nb], gf[x0:x0 + nb, :]) + gy[nb]
        bfc = bf_r[0, x0 * NP:(x0 + nb) * NP, :]
        ph = jnp.maximum(
            _dot(lp, Wpl_r[...]) + _dot(gp, Wpg_r[...])
            + _dot(bfc, Wpb_r[...]) + bp_r[...], 0.0)
        sc = _dot(ph, Wsc_r[...]) + bsc_r[...]
        out_r[0, x0 * NP:(x0 + nb) * NP, :] = sc

    scv = out_r[0]
    blv = bl_r[0]
    I_r = _iota((NNP, 1), 0)
    xid = jnp.floor_divide(I_r, NP)
    valid = (I_r - xid * NP) < N
    refidx = (I_r - 4 * xid) * C + _iota((NNP, C), 1)
    mskv = jnp.where(blv == -1.0, scv - 10000.0, scv)
    mskv = jnp.where(valid, mskv, NEG)
    I = jnp.where(valid, refidx, jnp.int32(2**31 - 1))
    lane = _iota((1, K), 1)
    row = jnp.zeros((1, K), jnp.int32)
    for k in range(K):
        m = jnp.max(mskv)
        cand = jnp.where(mskv == m, I, jnp.int32(2**31 - 1))
        ji = jnp.min(cand)
        row = jnp.where(lane == k, ji, row)
        mskv = jnp.where(I == ji, NEG, mskv)
    tk_r[0] = row


def kernel(fatoms, fbonds, atom_nb, bond_nb, num_nbs, n_atoms, binary_feats,
           blabels, mask_neis, mask_atoms,
           W_embed, b_embed, W_nei_atom, W_nei_bond, W_self, W_u2, W_u1,
           W_att_atom, W_att_bin, W_att_score,
           W_pair_local, W_pair_global, W_pair_bin, b_pair, W_score, b_score):
    B = fatoms.shape[0]
    padn = [(0, 0), (0, 0), (0, NP - N)]

    def km(x):
        return jnp.pad(jnp.transpose(x, (0, 2, 1)), padn).reshape(B, RK, 1)

    anb3 = km(atom_nb.astype(jnp.int32))
    bnb3 = km(bond_nb.astype(jnp.int32))
    mn3 = km(mask_neis[..., 0])
    bf_p = jnp.pad(binary_feats,
                   [(0, 0), (0, 0), (0, NP - N), (0, 0)]).reshape(B, NNP, BIN)
    bl_p = jnp.pad(blabels,
                   [(0, 0), (0, 0), (0, NP - N), (0, 0)]).reshape(B, NNP, C)
    ma2 = mask_atoms.reshape(B, N, 1)
    be2 = b_embed.reshape(1, H)
    bp2 = b_pair.reshape(1, H)
    bsc2 = b_score.reshape(1, C)

    def im3(b):
        return (b, 0, 0)

    def im2(b):
        return (0, 0)

    def imw3(b):
        return (0, 0, 0)

    in_specs = [
        pl.BlockSpec((1, N, AF), im3),
        pl.BlockSpec((1, M, BF), im3),
        pl.BlockSpec((1, RK, 1), im3),
        pl.BlockSpec((1, RK, 1), im3),
        pl.BlockSpec((1, NNP, BIN), im3),
        pl.BlockSpec((1, NNP, C), im3),
        pl.BlockSpec((1, RK, 1), im3),
        pl.BlockSpec((1, N, 1), im3),
        pl.BlockSpec((AF, H), im2),
        pl.BlockSpec((1, H), im2),
        pl.BlockSpec((DEPTH, H, H), imw3),
        pl.BlockSpec((DEPTH, BF, H), imw3),
        pl.BlockSpec((DEPTH, H, H), imw3),
        pl.BlockSpec((DEPTH, H + BF, H), imw3),
        pl.BlockSpec((DEPTH, 2 * H, H), imw3),
        pl.BlockSpec((H, H), im2),
        pl.BlockSpec((BIN, H), im2),
        pl.BlockSpec((H, 1), im2),
        pl.BlockSpec((H, H), im2),
        pl.BlockSpec((H, H), im2),
        pl.BlockSpec((BIN, H), im2),
        pl.BlockSpec((1, H), im2),
        pl.BlockSpec((H, C), im2),
        pl.BlockSpec((1, C), im2),
    ]
    out_specs = [
        pl.BlockSpec((1, NNP, C), im3),
        pl.BlockSpec((1, 1, K), im3),
    ]
    out_shapes = [
        jax.ShapeDtypeStruct((B, NNP, C), jnp.float32),
        jax.ShapeDtypeStruct((B, 1, K), jnp.int32),
    ]
    scores_p, topk3 = pl.pallas_call(
        _body,
        grid=(B,),
        in_specs=in_specs,
        out_specs=out_specs,
        out_shape=out_shapes,
    )(fatoms, fbonds, anb3, bnb3, bf_p, bl_p, mn3, ma2,
      W_embed, be2, W_nei_atom, W_nei_bond, W_self, W_u2, W_u1,
      W_att_atom, W_att_bin, W_att_score,
      W_pair_local, W_pair_global, W_pair_bin, bp2, W_score, bsc2)
    pair_scores = scores_p.reshape(B, N, NP, C)[:, :, :N, :]
    return (pair_scores, topk3.reshape(B, K))


# speedup vs baseline: 1.0000x; 1.0000x over previous
import jax
import jax.numpy as jnp
from jax.experimental import pallas as pl
from jax.experimental.pallas import tpu as pltpu

DEPTH = 3
AF = 82
BF = 6
H = 300
BIN = 11
N = 100
M = 120
NB = 10
C = 5
NP = 104
RK = NB * NP
NNP = N * NP
K = 20
NEG = -3e38

_CHUNKS = [(x0, 8) for x0 in range(0, 96, 8)] + [(96, 4)]


def _dot(a, b):
    return jnp.dot(a, b, preferred_element_type=jnp.float32)


def _iota(shape, dim):
    return jax.lax.broadcasted_iota(jnp.int32, shape, dim)


def _egather(oh, tbl):
    bits = jax.lax.bitcast_convert_type(tbl, jnp.int32)
    acc = None
    for sh in (0, 8, 16, 24):
        byte = jnp.right_shift(bits, sh) & 0xFF
        plane = _dot(oh, byte.astype(jnp.float32)).astype(jnp.int32)
        term = jnp.left_shift(plane, sh)
        acc = term if acc is None else acc | term
    return jax.lax.bitcast_convert_type(acc, jnp.float32)


def _band_onehot(nbands):
    rows = nbands * NP
    r = _iota((rows, nbands), 0)
    c = _iota((rows, nbands), 1)
    return ((r >= c * NP) & (r < c * NP + NP)).astype(jnp.float32)


def _band_sum(tp, off):
    acc = tp[off:off + 8, :]
    for j in range(1, 13):
        acc = acc + tp[off + 8 * j:off + 8 * (j + 1), :]
    for st in (4, 2, 1):
        acc = acc[:st, :] + acc[st:2 * st, :]
    return acc


def _body(fat_r, fbd_r, anb_r, bnb_r, bf_r, bl_r, mn_r, ma_r,
          We_r, be_r, Wna_r, Wnb_r, Ws_r, Wu2_r, Wu1_r,
          Waa_r, Wab_r, Was_r, Wpl_r, Wpg_r, Wpb_r, bp_r, Wsc_r, bsc_r,
          out_r, tk_r):
    f32 = jnp.float32
    fat = fat_r[0]
    fbd = fbd_r[0]
    anb = anb_r[0]
    bnb = bnb_r[0]
    mnei = mn_r[0]
    mat = ma_r[0]

    oh_a = (anb == _iota((RK, N), 1)).astype(f32)
    oh_b = (bnb == _iota((RK, M), 1)).astype(f32)
    fb = _egather(oh_b, fbd)

    atom_feat = jnp.maximum(_dot(fat, We_r[...]) + be_r[...], 0.0)
    local = atom_feat
    for d in range(DEPTH):
        fnei = _egather(oh_a, atom_feat)
        hm = _dot(fnei, Wna_r[d]) * _dot(fb, Wnb_r[d]) * mnei
        f_nei = hm[0:N, :]
        for k in range(1, NB):
            f_nei = f_nei + hm[k * NP:k * NP + N, :]
        f_self = _dot(atom_feat, Ws_r[d])
        local = f_nei * f_self * mat
        npm = jnp.maximum(_dot(jnp.concatenate([fnei, fb], axis=1),
                               Wu2_r[d]), 0.0) * mnei
        nei_partial = npm[0:N, :]
        for k in range(1, NB):
            nei_partial = nei_partial + npm[k * NP:k * NP + N, :]
        atom_feat = jnp.maximum(
            _dot(jnp.concatenate([atom_feat, nei_partial], axis=1),
                 Wu1_r[d]), 0.0)

    a = _dot(local, Waa_r[...])
    zpad = jnp.zeros((NP - N, H), f32)
    a104 = jnp.concatenate([a, zpad], axis=0)
    l104 = jnp.concatenate([local, zpad], axis=0)
    boh = {nb: _band_onehot(nb) for nb in (8, 4)}
    ay = {nb: jnp.concatenate([a104] * nb, axis=0) for nb in (8, 4)}
    lt = {nb: jnp.concatenate([l104] * nb, axis=0) for nb in (8, 4)}

    gf_parts = []
    for x0, nb in _CHUNKS:
        ax = _egather(boh[nb], a[x0:x0 + nb, :])
        bfc = bf_r[0, x0 * NP:(x0 + nb) * NP, :]
        hid = jnp.maximum(_dot(bfc, Wab_r[...]) + (ax + ay[nb]), 0.0)
        logit = _dot(hid, Was_r[...])
        s = 1.0 / (1.0 + jnp.exp(-logit))
        tp = s * lt[nb]
        for b in range(nb):
            gf_parts.append(_band_sum(tp, b * NP))
    gf = jnp.concatenate(gf_parts, axis=0)
    g104 = jnp.concatenate([gf, zpad], axis=0)
    gy = {nb: jnp.concatenate([g104] * nb, axis=0) for nb in (8, 4)}

    for x0, nb in _CHUNKS:
        lp = _egather(boh[nb], local[x0:x0 + nb, :]) + lt[nb]
        gp = _egather(boh[nb], gf[x0:x0 + nb, :]) + gy[nb]
        bfc = bf_r[0, x0 * NP:(x0 + nb) * NP, :]
        ph = jnp.maximum(
            _dot(lp, Wpl_r[...]) + _dot(gp, Wpg_r[...])
            + _dot(bfc, Wpb_r[...]) + bp_r[...], 0.0)
        sc = _dot(ph, Wsc_r[...]) + bsc_r[...]
        out_r[0, x0 * NP:(x0 + nb) * NP, :] = sc

    scv = out_r[0]
    blv = bl_r[0]
    I_r = _iota((NNP, 1), 0)
    xid = jnp.floor_divide(I_r, NP)
    valid = (I_r - xid * NP) < N
    refidx = (I_r - 4 * xid) * C + _iota((NNP, C), 1)
    mskv = jnp.where(blv == -1.0, scv - 10000.0, scv)
    mskv = jnp.where(valid, mskv, NEG)
    I = jnp.where(valid, refidx, jnp.int32(2**31 - 1))
    lane = _iota((1, K), 1)
    row = jnp.zeros((1, K), jnp.int32)
    for k in range(K):
        m = jnp.max(mskv)
        cand = jnp.where(mskv == m, I, jnp.int32(2**31 - 1))
        ji = jnp.min(cand)
        row = jnp.where(lane == k, ji, row)
        mskv = jnp.where(I == ji, NEG, mskv)
    tk_r[0] = row


def kernel(fatoms, fbonds, atom_nb, bond_nb, num_nbs, n_atoms, binary_feats,
           blabels, mask_neis, mask_atoms,
           W_embed, b_embed, W_nei_atom, W_nei_bond, W_self, W_u2, W_u1,
           W_att_atom, W_att_bin, W_att_score,
           W_pair_local, W_pair_global, W_pair_bin, b_pair, W_score, b_score):
    B = fatoms.shape[0]
    padn = [(0, 0), (0, 0), (0, NP - N)]

    def km(x):
        return jnp.pad(jnp.transpose(x, (0, 2, 1)), padn).reshape(B, RK, 1)

    anb3 = km(atom_nb.astype(jnp.int32))
    bnb3 = km(bond_nb.astype(jnp.int32))
    mn3 = km(mask_neis[..., 0])
    bf_p = jnp.pad(binary_feats,
                   [(0, 0), (0, 0), (0, NP - N), (0, 0)]).reshape(B, NNP, BIN)
    bl_p = jnp.pad(blabels,
                   [(0, 0), (0, 0), (0, NP - N), (0, 0)]).reshape(B, NNP, C)
    ma2 = mask_atoms.reshape(B, N, 1)
    be2 = b_embed.reshape(1, H)
    bp2 = b_pair.reshape(1, H)
    bsc2 = b_score.reshape(1, C)

    def im3(b):
        return (b, 0, 0)

    def im2(b):
        return (0, 0)

    def imw3(b):
        return (0, 0, 0)

    in_specs = [
        pl.BlockSpec((1, N, AF), im3),
        pl.BlockSpec((1, M, BF), im3),
        pl.BlockSpec((1, RK, 1), im3),
        pl.BlockSpec((1, RK, 1), im3),
        pl.BlockSpec((1, NNP, BIN), im3),
        pl.BlockSpec((1, NNP, C), im3),
        pl.BlockSpec((1, RK, 1), im3),
        pl.BlockSpec((1, N, 1), im3),
        pl.BlockSpec((AF, H), im2),
        pl.BlockSpec((1, H), im2),
        pl.BlockSpec((DEPTH, H, H), imw3),
        pl.BlockSpec((DEPTH, BF, H), imw3),
        pl.BlockSpec((DEPTH, H, H), imw3),
        pl.BlockSpec((DEPTH, H + BF, H), imw3),
        pl.BlockSpec((DEPTH, 2 * H, H), imw3),
        pl.BlockSpec((H, H), im2),
        pl.BlockSpec((BIN, H), im2),
        pl.BlockSpec((H, 1), im2),
        pl.BlockSpec((H, H), im2),
        pl.BlockSpec((H, H), im2),
        pl.BlockSpec((BIN, H), im2),
        pl.BlockSpec((1, H), im2),
        pl.BlockSpec((H, C), im2),
        pl.BlockSpec((1, C), im2),
    ]
    out_specs = [
        pl.BlockSpec((1, NNP, C), im3),
        pl.BlockSpec((1, 1, K), im3),
    ]
    out_shapes = [
        jax.ShapeDtypeStruct((B, NNP, C), jnp.float32),
        jax.ShapeDtypeStruct((B, 1, K), jnp.int32),
    ]
    scores_p, topk3 = pl.pallas_call(
        _body,
        grid=(B,),
        in_specs=in_specs,
        out_specs=out_specs,
        out_shape=out_shapes,
    )(fatoms, fbonds, anb3, bnb3, bf_p, bl_p, mn3, ma2,
      W_embed, be2, W_nei_atom, W_nei_bond, W_self, W_u2, W_u1,
      W_att_atom, W_att_bin, W_att_score,
      W_pair_local, W_pair_global, W_pair_bin, bp2, W_score, bsc2)
    pair_scores = scores_p.reshape(B, N, NP, C)[:, :, :N, :]
    return (pair_scores, topk3.reshape(B, K))
